# baseline (speedup 1.0000x reference)
import jax
import jax.numpy as jnp
from jax import lax
from jax.experimental import pallas as pl
from jax.experimental.pallas import tpu as pltpu
from jax.experimental.pallas import tpu_sc as plsc

NC = 2
NS = 16
NW = NC * NS
LANES = 16

W = 20
E = 128
N_UNITS = 1024 * 20 + 1024
UNITS_PER_WORKER = N_UNITS // NW
CHUNK = 6
N_CHUNKS = UNITS_PER_WORKER // CHUNK
ROWS = CHUNK * W
N_STEPS = N_CHUNKS // 2
HALF_CHUNKS = N_CHUNKS // 2
STAGE_UNITS = HALF_CHUNKS * CHUNK
IDX_PER_WORKER = (N_CHUNKS + 1) * ROWS


def _gather_start(table_hbm, idx_v, c, rows_buf, sem):
    pltpu.async_copy(table_hbm.at[idx_v.at[pl.ds(c * ROWS, ROWS)]],
                     rows_buf, sem)


def _gather_wait(table_hbm, idx_v, c, rows_buf, sem):
    pltpu.make_async_copy(table_hbm.at[idx_v.at[pl.ds(c * ROWS, ROWS)]],
                          rows_buf, sem).wait()


def _compute_chunk(rows_buf, pos_v, stage_v, slot):
    for j in range(E // LANES):
        col = pl.ds(j * LANES, LANES)

        def w_body(w, accs):
            p = pos_v[w, col]
            return tuple(accs[s] + rows_buf[s * W + w, col] * p
                         for s in range(CHUNK))

        zero = jnp.zeros((LANES,), jnp.float32)
        accs = lax.fori_loop(0, W, w_body, tuple(zero for _ in range(CHUNK)))
        for s in range(CHUNK):
            stage_v[pl.ds((slot + s) * E + j * LANES, LANES)] = accs[s]


def _wsum_body(idx_hbm, pos_hbm, table_hbm, out_hbm, idx_v, pos_v, rows0,
               rows1, stage_v, gsem0, gsem1):
    cid = lax.axis_index("c")
    sid = lax.axis_index("s")
    wid = sid * NC + cid
    pltpu.sync_copy(idx_hbm.at[pl.ds(wid * IDX_PER_WORKER, IDX_PER_WORKER)],
                    idx_v)
    pltpu.sync_copy(pos_hbm, pos_v)
    base = wid * UNITS_PER_WORKER * E

    _gather_start(table_hbm, idx_v, 0, rows0, gsem0)

    def body(g, carry):
        c0 = 2 * g
        slot0 = lax.rem(c0, HALF_CHUNKS) * CHUNK
        _gather_start(table_hbm, idx_v, c0 + 1, rows1, gsem1)
        _gather_wait(table_hbm, idx_v, c0, rows0, gsem0)
        _compute_chunk(rows0, pos_v, stage_v, slot0)
        _gather_start(table_hbm, idx_v, c0 + 2, rows0, gsem0)
        _gather_wait(table_hbm, idx_v, c0 + 1, rows1, gsem1)
        _compute_chunk(rows1, pos_v, stage_v, slot0 + CHUNK)

        @pl.when(g == N_STEPS // 2 - 1)
        def _():
            pltpu.sync_copy(stage_v,
                            out_hbm.at[pl.ds(base, STAGE_UNITS * E)])

        @pl.when(g == N_STEPS - 1)
        def _():
            pltpu.sync_copy(
                stage_v,
                out_hbm.at[pl.ds(base + STAGE_UNITS * E, STAGE_UNITS * E)])

        return carry

    lax.fori_loop(0, N_STEPS, body, 0)
    _gather_wait(table_hbm, idx_v, N_CHUNKS, rows0, gsem0)


@jax.jit
def _run(idx_all, pos, table):
    mesh = plsc.VectorSubcoreMesh(core_axis_name="c", subcore_axis_name="s",
                                  num_cores=NC, num_subcores=NS)
    k = pl.kernel(
        _wsum_body,
        out_type=jax.ShapeDtypeStruct((N_UNITS * E,), jnp.float32),
        mesh=mesh,
        scratch_types=[
            pltpu.VMEM((IDX_PER_WORKER,), jnp.int32),
            pltpu.VMEM((W, E), jnp.float32),
            pltpu.VMEM((ROWS, E), jnp.float32),
            pltpu.VMEM((ROWS, E), jnp.float32),
            pltpu.VMEM((STAGE_UNITS * E,), jnp.float32),
            pltpu.SemaphoreType.DMA,
            pltpu.SemaphoreType.DMA,
        ],
    )
    return k(idx_all, pos, table)


def kernel(story, query, word_table, pos_embed):
    b, s, w = story.shape
    idx_all = jnp.concatenate(
        [story.reshape(b * s, w), query], axis=0).reshape(NW, -1)
    pad = jnp.zeros((NW, ROWS), jnp.int32)
    idx_all = jnp.concatenate([idx_all, pad], axis=1).reshape(-1)
    out = _run(idx_all, pos_embed[:w], word_table)
    out = out.reshape(N_UNITS, E)
    sentence_sum = out[:b * s].reshape(b, s, E)
    query_sum = out[b * s:]
    return sentence_sum, query_sum
